# baseline (speedup 1.0000x reference)
import jax
import jax.numpy as jnp
from jax.experimental import pallas as pl
from jax.experimental.pallas import tpu as pltpu

_N = 10000
_NP = 10240
_TM1 = 384
_TMH = 640
_TM = 2 * _TMH
_F32 = jnp.float32
_BF16 = jnp.bfloat16
_PARAMS = pltpu.CompilerParams(vmem_limit_bytes=120 * 1024 * 1024)


def _lrelu(y):
    return jnp.where(y > 0, y, 0.01 * y)


def _sup1_body(x_ref, w_ref, o_ref):
    o_ref[...] = jnp.dot(
        x_ref[...], w_ref[...], preferred_element_type=_F32
    ).astype(_BF16)


def _layer1_body(a_ref, s_ref, w_ref, b_ref, a16_ref, sup_ref):
    a16 = a_ref[...].astype(_BF16)
    a16_ref[...] = a16
    y = jnp.dot(a16, s_ref[...], preferred_element_type=_F32) + b_ref[...]
    h = _lrelu(y)
    sup_ref[...] = jnp.dot(h, w_ref[...], preferred_element_type=_F32).astype(_BF16)


def _mid_body(a0_ref, a1_ref, s_ref, w_ref, b_ref, sup_ref):
    s, w, b = s_ref[...], w_ref[...], b_ref[...]
    for k, a_ref in enumerate((a0_ref, a1_ref)):
        y = jnp.dot(a_ref[0, 0], s, preferred_element_type=_F32) + b
        h = _lrelu(y)
        sup_ref[k * _TMH:(k + 1) * _TMH, :] = jnp.dot(
            h, w, preferred_element_type=_F32
        ).astype(_BF16)


def _lat_body(a0_ref, a1_ref, s_ref, w_ref, b_ref, lat_ref, sup_ref):
    s, w, b = s_ref[...], w_ref[...], b_ref[...]
    for k, a_ref in enumerate((a0_ref, a1_ref)):
        y = jnp.dot(a_ref[0, 0], s, preferred_element_type=_F32) + b
        lat_ref[k * _TMH:(k + 1) * _TMH, :] = y
        sup_ref[k * _TMH:(k + 1) * _TMH, :] = jnp.dot(
            y, w, preferred_element_type=_F32
        ).astype(_BF16)


def _last_body(a0_ref, a1_ref, s_ref, b_ref, out_ref):
    s, b = s_ref[...], b_ref[...]
    for k, a_ref in enumerate((a0_ref, a1_ref)):
        out_ref[k * _TMH:(k + 1) * _TMH, :] = (
            jnp.dot(a_ref[0, 0], s, preferred_element_type=_F32) + b
        )


def _row_spec(tm, d):
    return pl.BlockSpec((tm, d), lambda i: (i, 0))


def _half_spec(k):
    return pl.BlockSpec((1, 1, _TMH, _N), lambda i, k=k: (i, k, 0, 0))


def _full_spec(r, c):
    return pl.BlockSpec((r, c), lambda i: (0, 0))


def kernel(x, adj, inv_adj, W1, b1, W2, b2, W3, b3, W4, b4, W5, b5, W6, b6,
           W7, b7, W8, b8):
    del inv_adj
    n, d0 = x.shape
    ws = [W1, W2, W3, W4, W5, W6, W7, W8]
    bs = [b.reshape(1, -1) for b in (b1, b2, b3, b4, b5, b6, b7, b8)]
    dims = [d0] + [w.shape[1] for w in ws]

    sup = pl.pallas_call(
        _sup1_body,
        grid=(pl.cdiv(n, 800),),
        in_specs=[_row_spec(800, d0), _full_spec(d0, dims[1])],
        out_specs=_row_spec(800, dims[1]),
        out_shape=jax.ShapeDtypeStruct((n, dims[1]), _BF16),
        compiler_params=_PARAMS,
    )(x, W1)

    adj16, sup = pl.pallas_call(
        _layer1_body,
        grid=(pl.cdiv(_NP, _TM1),),
        in_specs=[
            _row_spec(_TM1, n),
            _full_spec(n, dims[1]),
            _full_spec(dims[1], dims[2]),
            _full_spec(1, dims[1]),
        ],
        out_specs=[_row_spec(_TM1, n), _row_spec(_TM1, dims[2])],
        out_shape=[
            jax.ShapeDtypeStruct((_NP, n), _BF16),
            jax.ShapeDtypeStruct((n, dims[2]), _BF16),
        ],
        compiler_params=_PARAMS,
    )(adj, sup, W2, bs[0])

    adj16v = adj16.reshape(_NP // _TM, 2, _TMH, n)

    def gc_layer(body, sup_d, out_specs, out_shape, *args):
        return pl.pallas_call(
            body,
            grid=(_NP // _TM,),
            in_specs=[_half_spec(0), _half_spec(1), _full_spec(n, sup_d)]
            + [_full_spec(*a.shape) for a in args],
            out_specs=out_specs,
            out_shape=out_shape,
            compiler_params=_PARAMS,
        )

    for li in (2, 3):
        sup = gc_layer(
            _mid_body, dims[li],
            _row_spec(_TM, dims[li + 1]),
            jax.ShapeDtypeStruct((n, dims[li + 1]), _BF16),
            ws[li], bs[li - 1],
        )(adj16v, adj16v, sup, ws[li], bs[li - 1])

    lat, sup = gc_layer(
        _lat_body, dims[4],
        [_row_spec(_TM, dims[4]), _row_spec(_TM, dims[5])],
        [
            jax.ShapeDtypeStruct((n, dims[4]), _F32),
            jax.ShapeDtypeStruct((n, dims[5]), _BF16),
        ],
        ws[4], bs[3],
    )(adj16v, adj16v, sup, W5, bs[3])

    for li in (5, 6, 7):
        sup = gc_layer(
            _mid_body, dims[li],
            _row_spec(_TM, dims[li + 1]),
            jax.ShapeDtypeStruct((n, dims[li + 1]), _BF16),
            ws[li], bs[li - 1],
        )(adj16v, adj16v, sup, ws[li], bs[li - 1])

    out = gc_layer(
        _last_body, dims[8],
        _row_spec(_TM, dims[8]),
        jax.ShapeDtypeStruct((n, dims[8]), _F32),
        bs[7],
    )(adj16v, adj16v, sup, bs[7])

    return (lat, out)


# speedup vs baseline: 1.4681x; 1.4681x over previous
import jax
import jax.numpy as jnp
from jax.experimental import pallas as pl
from jax.experimental.pallas import tpu as pltpu

_N = 10000
_TM1 = 400
_TM = 400
_NBLK = _N // _TM
_F32 = jnp.float32
_BF16 = jnp.bfloat16
_PARAMS = pltpu.CompilerParams(vmem_limit_bytes=120 * 1024 * 1024)

_STREAM_SPEC = pl.BlockSpec(
    (_TM, _N), lambda i: (i, 0), pipeline_mode=pl.Buffered(buffer_count=4)
)


def _lrelu(y):
    return jnp.where(y > 0, y, 0.01 * y)


def _sup1_body(x_ref, w_ref, o_ref):
    o_ref[...] = jnp.dot(
        x_ref[...], w_ref[...], preferred_element_type=_F32
    ).astype(_BF16)


def _layer1_body(a_ref, s_ref, w_ref, b_ref, a16_ref, sup_ref):
    a16 = a_ref[...].astype(_BF16)
    a16_ref[...] = a16
    y = jnp.dot(a16, s_ref[...], preferred_element_type=_F32) + b_ref[...]
    h = _lrelu(y)
    sup_ref[...] = jnp.dot(h, w_ref[...], preferred_element_type=_F32).astype(_BF16)


def _deep_body(adj_ref, s2_ref, w3_ref, w4_ref, w5_ref, w6_ref, w7_ref, w8_ref,
               b2_ref, b3_ref, b4_ref, b5_ref, b6_ref, b7_ref, b8_ref,
               lat_ref, out_ref, supa_ref, supb_ref, cnt_ref):

    def run_layer(step):
        cnt_ref[0] = 0

        def inner(a_ref):
            i = cnt_ref[0]
            cnt_ref[0] = i + 1
            step(a_ref[...], pl.ds(i * _TM, _TM))

        pltpu.emit_pipeline(
            inner, grid=(_NBLK,), in_specs=[_STREAM_SPEC]
        )(adj_ref)

    def l2(a, rows):
        h = _lrelu(jnp.dot(a, s2_ref[...], preferred_element_type=_F32) + b2_ref[...])
        supa_ref[rows, :16] = jnp.dot(h, w3_ref[...], preferred_element_type=_F32).astype(_BF16)

    def l3(a, rows):
        h = _lrelu(jnp.dot(a, supa_ref[:, :16], preferred_element_type=_F32) + b3_ref[...])
        supb_ref[rows, :8] = jnp.dot(h, w4_ref[...], preferred_element_type=_F32).astype(_BF16)

    def l4(a, rows):
        y = jnp.dot(a, supb_ref[:, :8], preferred_element_type=_F32) + b4_ref[...]
        lat_ref[rows, :] = y
        supa_ref[rows, :16] = jnp.dot(y, w5_ref[...], preferred_element_type=_F32).astype(_BF16)

    def l5(a, rows):
        h = _lrelu(jnp.dot(a, supa_ref[:, :16], preferred_element_type=_F32) + b5_ref[...])
        supb_ref[rows, :32] = jnp.dot(h, w6_ref[...], preferred_element_type=_F32).astype(_BF16)

    def l6(a, rows):
        h = _lrelu(jnp.dot(a, supb_ref[:, :32], preferred_element_type=_F32) + b6_ref[...])
        supa_ref[rows, :64] = jnp.dot(h, w7_ref[...], preferred_element_type=_F32).astype(_BF16)

    def l7(a, rows):
        h = _lrelu(jnp.dot(a, supa_ref[:, :64], preferred_element_type=_F32) + b7_ref[...])
        supb_ref[rows, :] = jnp.dot(h, w8_ref[...], preferred_element_type=_F32).astype(_BF16)

    def l8(a, rows):
        out_ref[rows, :] = jnp.dot(a, supb_ref[...], preferred_element_type=_F32) + b8_ref[...]

    for step in (l2, l3, l4, l5, l6, l7, l8):
        run_layer(step)


def _row_spec(tm, d):
    return pl.BlockSpec((tm, d), lambda i: (i, 0))


def _full_spec(r, c):
    return pl.BlockSpec((r, c), lambda i: (0, 0))


def kernel(x, adj, inv_adj, W1, b1, W2, b2, W3, b3, W4, b4, W5, b5, W6, b6,
           W7, b7, W8, b8):
    del inv_adj
    n, d0 = x.shape
    bs = [b.reshape(1, -1) for b in (b1, b2, b3, b4, b5, b6, b7, b8)]

    sup1 = pl.pallas_call(
        _sup1_body,
        grid=(pl.cdiv(n, 800),),
        in_specs=[_row_spec(800, d0), _full_spec(d0, 64)],
        out_specs=_row_spec(800, 64),
        out_shape=jax.ShapeDtypeStruct((n, 64), _BF16),
        compiler_params=_PARAMS,
    )(x, W1)

    adj16, sup2 = pl.pallas_call(
        _layer1_body,
        grid=(n // _TM1,),
        in_specs=[
            _row_spec(_TM1, n),
            _full_spec(n, 64),
            _full_spec(64, 32),
            _full_spec(1, 64),
        ],
        out_specs=[_row_spec(_TM1, n), _row_spec(_TM1, 32)],
        out_shape=[
            jax.ShapeDtypeStruct((n, n), _BF16),
            jax.ShapeDtypeStruct((n, 32), _BF16),
        ],
        compiler_params=_PARAMS,
    )(adj, sup1, W2, bs[0])

    vmem = pl.BlockSpec(memory_space=pltpu.VMEM)
    lat, out = pl.pallas_call(
        _deep_body,
        in_specs=[pl.BlockSpec(memory_space=pl.ANY)] + [vmem] * 14,
        out_specs=[vmem, vmem],
        out_shape=[
            jax.ShapeDtypeStruct((n, 8), _F32),
            jax.ShapeDtypeStruct((n, 128), _F32),
        ],
        scratch_shapes=[
            pltpu.VMEM((n, 64), _BF16),
            pltpu.VMEM((n, 128), _BF16),
            pltpu.SMEM((1,), jnp.int32),
        ],
        compiler_params=_PARAMS,
    )(adj16, sup2, W3, W4, W5, W6, W7, W8, *bs[1:])

    return (lat, out)
